# baseline (speedup 1.0000x reference)
import jax
import jax.numpy as jnp
from jax.experimental import pallas as pl


def _add_kernel(x_ref, w_ref, o_ref):
    o_ref[...] = x_ref[...] + w_ref[...][None, :, None, :]


def kernel(x, W):
    batch, n, t, f = x.shape
    nb = 400
    grid = (batch, n // nb)
    return pl.pallas_call(
        _add_kernel,
        grid=grid,
        in_specs=[
            pl.BlockSpec((1, nb, t, f), lambda b, i: (b, i, 0, 0)),
            pl.BlockSpec((nb, f), lambda b, i: (i, 0)),
        ],
        out_specs=pl.BlockSpec((1, nb, t, f), lambda b, i: (b, i, 0, 0)),
        out_shape=jax.ShapeDtypeStruct(x.shape, x.dtype),
    )(x, W)
